# Initial kernel scaffold; baseline (speedup 1.0000x reference)
import jax
import jax.numpy as jnp
from jax.experimental import pallas as pl


def kernel(x, edge_index, lin_w, att_i, att_j, struct_w, struct_b, bias, bn_gamma, bn_beta):
    raise NotImplementedError("write your pallas kernel here")


# speedup vs baseline: 19.7272x; 19.7272x over previous
import functools

import jax
import jax.numpy as jnp
from jax import lax
from jax.experimental import pallas as pl
from jax.experimental.pallas import tpu as pltpu
from jax.experimental.pallas import tpu_sc as plsc

_N = 10000
_E = 320000
_D = 128
_NEG = 0.2

_NC = 2
_NS = 16
_NW = _NC * _NS

_C = 128
_NCH = 81
_EW = _C * _NCH
_EP = _EW * _NW

_NA = 10240
_RPT = _NA // _NS


def _tc_pre(x_ref, w_ref, wi_ref, wj_ref, xl_ref, ai_ref, aj_ref):
    xl = jnp.dot(x_ref[...], w_ref[...], preferred_element_type=jnp.float32)
    xl_ref[...] = xl
    ai_ref[...] = jnp.dot(xl, wi_ref[...], preferred_element_type=jnp.float32)
    aj_ref[...] = jnp.dot(xl, wj_ref[...], preferred_element_type=jnp.float32)


def _sc_edges(xl_hbm, ai_hbm, aj_hbm, src_hbm, dstg_hbm, dste_hbm,
              acc_out, den_out,
              acc_sh, den_sh, ai_v, aj_v, src_c, dstg_c, dste_c,
              rows_v, p_c, sem):
    c = lax.axis_index("c")
    s = lax.axis_index("s")
    wid = c * _NS + s

    pltpu.sync_copy(ai_hbm, ai_v)
    pltpu.sync_copy(aj_hbm, aj_v)

    def _zrow(e, carry):
        for j in range(8):
            rows_v[e, pl.ds(j * 16, 16)] = jnp.zeros((16,), jnp.float32)
        return carry

    lax.fori_loop(0, _C, _zrow, 0)
    for j in range(_C // 16):
        p_c[pl.ds(j * 16, 16)] = jnp.zeros((16,), jnp.float32)

    r0 = s * _RPT
    for kk in range(_RPT // _C):
        pltpu.sync_copy(rows_v, acc_sh.at[pl.ds(r0 + kk * _C, _C)])
        pltpu.sync_copy(p_c, den_sh.at[pl.ds(r0 + kk * _C, _C)])

    plsc.subcore_barrier()

    def _chunk(k, carry):
        pltpu.sync_copy(src_hbm.at[wid].at[k], src_c)
        pltpu.sync_copy(dstg_hbm.at[wid].at[k], dstg_c)
        pltpu.sync_copy(dste_hbm.at[wid].at[k], dste_c)
        rows_dma = pltpu.async_copy(xl_hbm.at[src_c], rows_v, sem)

        def _pg(j, carry2):
            sidx = src_c[pl.ds(j * 16, 16)]
            didx = dstg_c[pl.ds(j * 16, 16)]
            a = plsc.load_gather(ai_v, [didx]) + plsc.load_gather(aj_v, [sidx])
            a = jnp.maximum(a, a * _NEG)
            p_c[pl.ds(j * 16, 16)] = jnp.exp(a)
            return carry2

        lax.fori_loop(0, _C // 16, _pg, 0)
        rows_dma.wait()

        def _scale(g, carry2):
            pv = p_c[pl.ds(g * 16, 16)]
            for e in range(16):
                pe = pv[e]
                row = g * 16 + e
                for j in range(8):
                    rows_v[row, pl.ds(j * 16, 16)] = (
                        rows_v[row, pl.ds(j * 16, 16)] * pe)
            return carry2

        lax.fori_loop(0, _C // 16, _scale, 0)
        pltpu.sync_copy(rows_v, acc_sh.at[dste_c], add=True)
        pltpu.sync_copy(p_c, den_sh.at[dste_c], add=True)
        return carry

    lax.fori_loop(0, _NCH, _chunk, 0)

    plsc.subcore_barrier()

    pltpu.sync_copy(acc_sh.at[pl.ds(r0, _RPT)], acc_out.at[c].at[pl.ds(r0, _RPT)])
    pltpu.sync_copy(den_sh.at[pl.ds(r0, _RPT)], den_out.at[c].at[pl.ds(r0, _RPT)])


_sc_edges_kernel = functools.partial(
    pl.kernel,
    out_type=[
        jax.ShapeDtypeStruct((_NC, _NA, _D), jnp.float32),
        jax.ShapeDtypeStruct((_NC, _NA), jnp.float32),
    ],
    mesh=plsc.VectorSubcoreMesh(core_axis_name="c", subcore_axis_name="s"),
    compiler_params=pltpu.CompilerParams(needs_layout_passes=False),
    scratch_types=[
        pltpu.VMEM_SHARED((_NA, _D), jnp.float32),
        pltpu.VMEM_SHARED((_NA,), jnp.float32),
        pltpu.VMEM((_N,), jnp.float32),
        pltpu.VMEM((_N,), jnp.float32),
        pltpu.VMEM((_C,), jnp.int32),
        pltpu.VMEM((_C,), jnp.int32),
        pltpu.VMEM((_C,), jnp.int32),
        pltpu.VMEM((_C, _D), jnp.float32),
        pltpu.VMEM((_C,), jnp.float32),
        pltpu.SemaphoreType.DMA,
    ],
)(_sc_edges)


def _tc_post(acc_ref, d0_ref, d1_ref, sw_ref, sb_ref, b_ref, g_ref, be_ref,
             y_ref):
    agg = acc_ref[0, :_N, :] + acc_ref[1, :_N, :]
    out = agg / (d0_ref[...] + d1_ref[...] + 1e-16)
    out2 = (jnp.dot(out, sw_ref[...], preferred_element_type=jnp.float32)
            + sb_ref[...] + b_ref[...])
    mu = jnp.mean(out2, axis=0, keepdims=True)
    var = jnp.mean((out2 - mu) ** 2, axis=0, keepdims=True)
    obn = (out2 - mu) * lax.rsqrt(var + 1e-5) * g_ref[...] + be_ref[...]
    y_ref[...] = out2 + jnp.maximum(obn, 0.0)


def kernel(x, edge_index, lin_w, att_i, att_j, struct_w, struct_b, bias,
           bn_gamma, bn_beta):
    src0 = edge_index[0].astype(jnp.int32)
    dst0 = edge_index[1].astype(jnp.int32)
    loops = jnp.arange(_N, dtype=jnp.int32)
    dste0 = jnp.where(src0 == dst0, _N, dst0)

    npad = _EP - (_E + _N)
    padi = jnp.arange(npad, dtype=jnp.int32)
    pad_src = (padi * 37) % _N
    pad_dste = _N + padi % (_NA - _N)

    src_all = jnp.concatenate([src0, loops, pad_src]).reshape(_NW, _NCH, _C)
    dstg_all = jnp.concatenate([dst0, loops, pad_src]).reshape(_NW, _NCH, _C)
    dste_all = jnp.concatenate([dste0, loops, pad_dste]).reshape(_NW, _NCH, _C)

    xl, ai, aj = pl.pallas_call(
        _tc_pre,
        out_shape=[
            jax.ShapeDtypeStruct((_N, _D), jnp.float32),
            jax.ShapeDtypeStruct((_N, 1), jnp.float32),
            jax.ShapeDtypeStruct((_N, 1), jnp.float32),
        ],
    )(x, lin_w, att_i.reshape(_D, 1), att_j.reshape(_D, 1))

    acc, den = _sc_edges_kernel(xl, ai.reshape(_N), aj.reshape(_N),
                                src_all, dstg_all, dste_all)

    d0 = den[0, :_N].reshape(_N, 1)
    d1 = den[1, :_N].reshape(_N, 1)
    y = pl.pallas_call(
        _tc_post,
        out_shape=jax.ShapeDtypeStruct((_N, _D), jnp.float32),
    )(acc, d0, d1, struct_w, struct_b.reshape(1, _D), bias.reshape(1, _D),
      bn_gamma.reshape(1, _D), bn_beta.reshape(1, _D))
    return y
